# Initial kernel scaffold; baseline (speedup 1.0000x reference)
import jax
import jax.numpy as jnp
from jax.experimental import pallas as pl


def kernel(positions, node_attrs, edge_index, shifts, atomic_energies, W_emb, W_rad1, W_mix1, W_rad2, W_mix2, W_read1, W_mlp1, W_mlp2):
    raise NotImplementedError("write your pallas kernel here")


# speedup vs baseline: 1.9715x; 1.9715x over previous
import functools

import jax
import jax.numpy as jnp
from jax import lax
from jax.experimental import pallas as pl
from jax.experimental.pallas import tpu as pltpu
from jax.experimental.pallas import tpu_sc as plsc

N = 50000
E = 800000
NUM_ELEMENTS = 10
H = 32
NUM_BESSEL = 8
P_CUT = 6
R_MAX = 5.0
SH_DIM = 4
AVG_NEIGH = 16.0
MLP_H = 16

NC = 2
NS = 16
NW = NC * NS
IDXB = 128
EPAD = 819200
ROWS_E = EPAD // IDXB
CH = 8
CHS = 4
NAGG = 50016
TILE_ROWS = NAGG // NS
OUT_ROWS = N // NS

BE = 3200
BN = 2000

_mesh = lambda: plsc.VectorSubcoreMesh(core_axis_name="c", subcore_axis_name="s")


def _make_gather(d, rows, n_tab):
    rows_w = rows // NW
    nch = rows_w // CH

    @functools.partial(
        pl.kernel,
        out_type=jax.ShapeDtypeStruct((rows * IDXB, d), jnp.float32),
        mesh=_mesh(),
        scratch_types=[
            pltpu.VMEM((CH, IDXB), jnp.int32),
            pltpu.VMEM((CH * IDXB, d), jnp.float32),
            pltpu.SemaphoreType.DMA,
        ],
        compiler_params=pltpu.CompilerParams(use_tc_tiling_on_sc=False),
    )
    def k(table, idx2, out, idxv, rowsv, sem):
        wid = lax.axis_index("s") * NC + lax.axis_index("c")
        rbase = wid * rows_w

        def body(i, carry):
            r0 = rbase + i * CH
            pltpu.sync_copy(idx2.at[pl.ds(r0, CH)], idxv)
            handles = []
            for j in range(CH):
                handles.append(pltpu.async_copy(
                    table.at[idxv.at[j]],
                    rowsv.at[pl.ds(j * IDXB, IDXB)], sem))
            for h2 in handles:
                h2.wait()
            pltpu.sync_copy(rowsv, out.at[pl.ds(r0 * IDXB, CH * IDXB)])
            return carry

        lax.fori_loop(0, nch, body, 0)

    return k


def _make_scatter():
    rows_w = ROWS_E // NW
    nch = rows_w // CHS

    @functools.partial(
        pl.kernel,
        out_type=jax.ShapeDtypeStruct((NC, N, H), jnp.float32),
        mesh=_mesh(),
        scratch_types=[
            pltpu.VMEM((CHS, IDXB), jnp.int32),
            pltpu.VMEM((CHS * IDXB, H), jnp.float32),
            pltpu.VMEM_SHARED((NAGG, H), jnp.float32),
            pltpu.SemaphoreType.DMA,
        ],
        compiler_params=pltpu.CompilerParams(use_tc_tiling_on_sc=False),
    )
    def k(v_hbm, idx2, zeros_hbm, out, idxv, vv, agg, sem):
        c = lax.axis_index("c")
        s = lax.axis_index("s")
        wid = s * NC + c
        pltpu.sync_copy(zeros_hbm, agg.at[pl.ds(s * TILE_ROWS, TILE_ROWS)])
        plsc.subcore_barrier()

        rbase = wid * rows_w

        def body(i, carry):
            r0 = rbase + i * CHS
            pltpu.sync_copy(idx2.at[pl.ds(r0, CHS)], idxv)
            pltpu.sync_copy(v_hbm.at[pl.ds(r0 * IDXB, CHS * IDXB)], vv)
            for j in range(CHS):
                pltpu.sync_copy(vv.at[pl.ds(j * IDXB, IDXB)],
                                agg.at[idxv.at[j]], add=True)
            return carry

        lax.fori_loop(0, nch, body, 0)
        plsc.subcore_barrier()
        pltpu.sync_copy(agg.at[pl.ds(s * OUT_ROWS, OUT_ROWS)],
                        out.at[c].at[pl.ds(s * OUT_ROWS, OUT_ROWS)])

    return k


def _embed_body(attrs_ref, wemb_ref, out_ref):
    out_ref[...] = jnp.dot(attrs_ref[...], wemb_ref[...],
                           preferred_element_type=jnp.float32)


def _geom_body(ps_ref, pr_ref, ea_ref, ef_ref):
    vec = pr_ref[:, 0:3] - ps_ref[:, 0:3]
    len2 = jnp.sum(vec * vec, axis=1, keepdims=True) + 1e-12
    lengths = jnp.sqrt(len2)
    inv = 1.0 / lengths
    unit = vec * inv
    ones = jnp.ones((vec.shape[0], 1), jnp.float32)
    ea_ref[...] = jnp.concatenate([ones, jnp.sqrt(3.0) * unit], axis=1)
    n = lax.broadcasted_iota(jnp.int32, (1, NUM_BESSEL), 1).astype(jnp.float32) + 1.0
    bessel = jnp.sqrt(2.0 / R_MAX) * jnp.sin(n * (jnp.pi / R_MAX) * lengths) * inv
    u = lengths * (1.0 / R_MAX)
    u2 = u * u
    u4 = u2 * u2
    u6 = u4 * u2
    u7 = u6 * u
    u8 = u7 * u
    p = float(P_CUT)
    env = (1.0
           - (p + 1.0) * (p + 2.0) / 2.0 * u6
           + p * (p + 2.0) * u7
           - p * (p + 1.0) / 2.0 * u8)
    env = jnp.where(u < 1.0, env, 0.0)
    ef_ref[...] = bessel * env


def _edge_body(g_ref, ea_ref, ef_ref, wrad_ref, wm_ref, v_ref):
    w = jnp.dot(ef_ref[...], wrad_ref[...], preferred_element_type=jnp.float32)
    u = g_ref[...] * w
    ea = ea_ref[...]
    acc = jnp.dot(u * ea[:, 0:1], wm_ref[0], preferred_element_type=jnp.float32)
    for s2 in range(1, SH_DIM):
        acc = acc + jnp.dot(u * ea[:, s2:s2 + 1], wm_ref[s2],
                            preferred_element_type=jnp.float32)
    v_ref[...] = acc


def _psum_body(p_ref, o_ref):
    o_ref[...] = (p_ref[0] + p_ref[1]) * (1.0 / AVG_NEIGH)


def _final_body(p2_ref, nf1_ref, attrs_ref, ae_ref, wr1_ref, wm1_ref, wm2_ref,
                o_ref):
    nf2 = (p2_ref[0] + p2_ref[1]) * (1.0 / AVG_NEIGH)
    e = jnp.dot(attrs_ref[...], ae_ref[...], preferred_element_type=jnp.float32)
    e = e + jnp.dot(nf1_ref[...], wr1_ref[...], preferred_element_type=jnp.float32)
    hpre = jnp.dot(nf2, wm1_ref[...], preferred_element_type=jnp.float32)
    hid = hpre * (1.0 / (1.0 + jnp.exp(-hpre)))
    e = e + jnp.dot(hid, wm2_ref[...], preferred_element_type=jnp.float32)
    o_ref[...] = e


def _whole(shape):
    return pl.BlockSpec(shape, lambda i: tuple(0 for _ in shape))


def kernel(positions, node_attrs, edge_index, shifts, atomic_energies, W_emb,
           W_rad1, W_mix1, W_rad2, W_mix2, W_read1, W_mlp1, W_mlp2):
    f32 = jnp.float32
    sender = edge_index[0]
    receiver = edge_index[1]
    pad = EPAD - E
    send_pad = jnp.concatenate([sender, jnp.zeros((pad,), jnp.int32)])
    recv_gpad = jnp.concatenate([receiver, jnp.zeros((pad,), jnp.int32)])
    recv_spad = jnp.concatenate([receiver, jnp.full((pad,), N, jnp.int32)])
    idx_pos = jnp.concatenate([send_pad, recv_gpad]).reshape(2 * ROWS_E, IDXB)
    idx_send = send_pad.reshape(ROWS_E, IDXB)
    idx_recv = recv_spad.reshape(ROWS_E, IDXB)
    pos16 = jnp.concatenate([positions, jnp.zeros((N, 13), f32)], axis=1)
    zeros_tile = jnp.zeros((TILE_ROWS, H), f32)

    poscat = _make_gather(16, 2 * ROWS_E, N)(pos16, idx_pos)
    ps = poscat[:EPAD]
    pr = poscat[EPAD:]

    n_eblk = EPAD // BE
    ea, ef = pl.pallas_call(
        _geom_body,
        grid=(n_eblk,),
        in_specs=[pl.BlockSpec((BE, 16), lambda i: (i, 0)),
                  pl.BlockSpec((BE, 16), lambda i: (i, 0))],
        out_specs=[pl.BlockSpec((BE, SH_DIM), lambda i: (i, 0)),
                   pl.BlockSpec((BE, NUM_BESSEL), lambda i: (i, 0))],
        out_shape=[jax.ShapeDtypeStruct((EPAD, SH_DIM), f32),
                   jax.ShapeDtypeStruct((EPAD, NUM_BESSEL), f32)],
    )(ps, pr)

    nf0 = pl.pallas_call(
        _embed_body,
        grid=(N // BN,),
        in_specs=[pl.BlockSpec((BN, NUM_ELEMENTS), lambda i: (i, 0)),
                  _whole((NUM_ELEMENTS, H))],
        out_specs=pl.BlockSpec((BN, H), lambda i: (i, 0)),
        out_shape=jax.ShapeDtypeStruct((N, H), f32),
    )(node_attrs, W_emb)

    gather32 = _make_gather(H, ROWS_E, N)
    scatter = _make_scatter()

    def edge_dense(g, wrad, wmix):
        wm = wmix.reshape(H, SH_DIM, H).transpose(1, 0, 2)
        return pl.pallas_call(
            _edge_body,
            grid=(n_eblk,),
            in_specs=[pl.BlockSpec((BE, H), lambda i: (i, 0)),
                      pl.BlockSpec((BE, SH_DIM), lambda i: (i, 0)),
                      pl.BlockSpec((BE, NUM_BESSEL), lambda i: (i, 0)),
                      _whole((NUM_BESSEL, H)),
                      _whole((SH_DIM, H, H))],
            out_specs=pl.BlockSpec((BE, H), lambda i: (i, 0)),
            out_shape=jax.ShapeDtypeStruct((EPAD, H), f32),
        )(g, ea, ef, wrad, wm)

    g1 = gather32(nf0, idx_send)
    v1 = edge_dense(g1, W_rad1, W_mix1)
    p1 = scatter(v1, idx_recv, zeros_tile)
    nf1 = pl.pallas_call(
        _psum_body,
        grid=(N // BN,),
        in_specs=[pl.BlockSpec((NC, BN, H), lambda i: (0, i, 0))],
        out_specs=pl.BlockSpec((BN, H), lambda i: (i, 0)),
        out_shape=jax.ShapeDtypeStruct((N, H), f32),
    )(p1)

    g2 = gather32(nf1, idx_send)
    v2 = edge_dense(g2, W_rad2, W_mix2)
    p2 = scatter(v2, idx_recv, zeros_tile)

    energies = pl.pallas_call(
        _final_body,
        grid=(N // BN,),
        in_specs=[pl.BlockSpec((NC, BN, H), lambda i: (0, i, 0)),
                  pl.BlockSpec((BN, H), lambda i: (i, 0)),
                  pl.BlockSpec((BN, NUM_ELEMENTS), lambda i: (i, 0)),
                  _whole((NUM_ELEMENTS, 1)),
                  _whole((H, 1)),
                  _whole((H, MLP_H)),
                  _whole((MLP_H, 1))],
        out_specs=pl.BlockSpec((BN, 1), lambda i: (i, 0)),
        out_shape=jax.ShapeDtypeStruct((N, 1), f32),
    )(p2, nf1, node_attrs, atomic_energies.reshape(NUM_ELEMENTS, 1),
      W_read1, W_mlp1, W_mlp2)
    return energies[:, 0]
